# baseline (speedup 1.0000x reference)
import functools

import jax
import jax.numpy as jnp
from jax.experimental import pallas as pl
from jax.experimental.pallas import tpu as pltpu

D = 2048
E = 8
N = 4096
TOP_K = 2

_BN = 1024
_BW = 256
_JW = D // _BW


def _gate_kernel(x_ref, gw_ref, gb_ref, idx_ref, w_ref):
    logits = jnp.dot(x_ref[...], gw_ref[...],
                     preferred_element_type=jnp.float32) + gb_ref[...]
    row = logits[0:1, :]
    m = jnp.max(row)
    e = jnp.exp(row - m)
    p = e / jnp.sum(e)
    lanes = jax.lax.broadcasted_iota(jnp.int32, (1, E), 1)
    m1 = jnp.max(p)
    a1 = jnp.min(jnp.where(p == m1, lanes, E))
    p2 = jnp.where(lanes == a1, -jnp.inf, p)
    m2 = jnp.max(p2)
    a2 = jnp.min(jnp.where(p2 == m2, lanes, E))
    idx_ref[...] = jnp.where(lanes == 0, a1, a2)
    w_ref[...] = jnp.where(lanes == 0, m1, m2)


def _mm_kernel(idx_ref, w_ref, x_ref, w0_ref, w1_ref, b0_ref, b1_ref, o_ref,
               wc_ref, bc_ref):
    del idx_ref
    t = pl.program_id(0)
    w0 = w_ref[0]
    w1 = w_ref[1]

    @pl.when(t < _JW)
    def _():
        wc_ref[:, pl.ds(t * _BW, _BW)] = (
            w0 * w0_ref[0] + w1 * w1_ref[0]).astype(jnp.bfloat16)

    @pl.when(t == 0)
    def _():
        bc_ref[...] = w0 * b0_ref[0] + w1 * b1_ref[0]

    @pl.when(t >= _JW)
    def _():
        acc = jnp.dot(x_ref[...].astype(jnp.bfloat16), wc_ref[...],
                      preferred_element_type=jnp.float32)
        o_ref[...] = acc + bc_ref[...]


@functools.partial(jax.jit, static_argnames=())
def kernel(x, gate_W, gate_b, experts_W, experts_b):
    idx8, w8 = pl.pallas_call(
        _gate_kernel,
        out_shape=[
            jax.ShapeDtypeStruct((1, E), jnp.int32),
            jax.ShapeDtypeStruct((1, E), jnp.float32),
        ],
    )(jax.lax.slice(x, (0, 0), (8, D)), gate_W, gate_b.reshape(1, E))
    idx2 = idx8[0, :TOP_K]
    wv = w8[0, :TOP_K]

    grid = (_JW + N // _BN,)
    out = pl.pallas_call(
        _mm_kernel,
        grid_spec=pltpu.PrefetchScalarGridSpec(
            num_scalar_prefetch=2,
            grid=grid,
            in_specs=[
                pl.BlockSpec((_BN, D),
                             lambda t, idx, w: (jnp.maximum(t - _JW, 0), 0)),
                pl.BlockSpec((1, D, _BW),
                             lambda t, idx, w: (idx[0], 0,
                                                jnp.minimum(t, _JW - 1))),
                pl.BlockSpec((1, D, _BW),
                             lambda t, idx, w: (idx[1], 0,
                                                jnp.minimum(t, _JW - 1))),
                pl.BlockSpec((1, 1, D), lambda t, idx, w: (idx[0], 0, 0)),
                pl.BlockSpec((1, 1, D), lambda t, idx, w: (idx[1], 0, 0)),
            ],
            out_specs=pl.BlockSpec((_BN, D),
                                   lambda t, idx, w: (jnp.maximum(t - _JW, 0),
                                                      0)),
            scratch_shapes=[pltpu.VMEM((D, D), jnp.bfloat16),
                            pltpu.VMEM((1, D), jnp.float32)],
        ),
        out_shape=jax.ShapeDtypeStruct((N, D), jnp.float32),
        compiler_params=pltpu.CompilerParams(
            dimension_semantics=("arbitrary",),
        ),
    )(idx2, wv, x, experts_W, experts_W,
      experts_b.reshape(E, 1, D), experts_b.reshape(E, 1, D))
    return out


# speedup vs baseline: 1.0694x; 1.0694x over previous
import functools

import jax
import jax.numpy as jnp
from jax.experimental import pallas as pl
from jax.experimental.pallas import tpu as pltpu

D = 2048
E = 8
N = 4096
TOP_K = 2

_BN = 512
_BW = 512
_JW = D // _BW


def _gate_kernel(x_ref, gw_ref, gb_ref, idx_ref, w_ref):
    logits = jnp.dot(x_ref[...], gw_ref[...],
                     preferred_element_type=jnp.float32) + gb_ref[...]
    row = logits[0:1, :]
    m = jnp.max(row)
    e = jnp.exp(row - m)
    p = e / jnp.sum(e)
    lanes = jax.lax.broadcasted_iota(jnp.int32, (1, E), 1)
    m1 = jnp.max(p)
    a1 = jnp.min(jnp.where(p == m1, lanes, E))
    p2 = jnp.where(lanes == a1, -jnp.inf, p)
    m2 = jnp.max(p2)
    a2 = jnp.min(jnp.where(p2 == m2, lanes, E))
    idx_ref[...] = jnp.where(lanes == 0, a1, a2)
    w_ref[...] = jnp.where(lanes == 0, m1, m2)


def _mm_kernel(idx_ref, w_ref, x_ref, w0_ref, w1_ref, b0_ref, b1_ref, o_ref,
               wc_ref, bc_ref):
    del idx_ref
    t = pl.program_id(0)
    w0 = w_ref[0, 0]
    w1 = w_ref[0, 1]

    @pl.when(t < _JW)
    def _():
        wc_ref[:, pl.ds(t * _BW, _BW)] = (
            w0 * w0_ref[0] + w1 * w1_ref[0]).astype(jnp.bfloat16)

    @pl.when(t == 0)
    def _():
        bc_ref[...] = w0 * b0_ref[0] + w1 * b1_ref[0]

    @pl.when(t >= _JW)
    def _():
        acc = jnp.dot(x_ref[...].astype(jnp.bfloat16), wc_ref[...],
                      preferred_element_type=jnp.float32)
        o_ref[...] = acc + bc_ref[...]


@functools.partial(jax.jit, static_argnames=())
def kernel(x, gate_W, gate_b, experts_W, experts_b):
    idx8, w8 = pl.pallas_call(
        _gate_kernel,
        out_shape=[
            jax.ShapeDtypeStruct((1, E), jnp.int32),
            jax.ShapeDtypeStruct((1, E), jnp.float32),
        ],
    )(jax.lax.slice(x, (0, 0), (8, D)), gate_W, gate_b.reshape(1, E))
    grid = (_JW + N // _BN,)
    out = pl.pallas_call(
        _mm_kernel,
        grid_spec=pltpu.PrefetchScalarGridSpec(
            num_scalar_prefetch=2,
            grid=grid,
            in_specs=[
                pl.BlockSpec((_BN, D),
                             lambda t, idx, w: (jnp.maximum(t - _JW, 0), 0)),
                pl.BlockSpec((1, D, _BW),
                             lambda t, idx, w: (idx[0, 0], 0,
                                                jnp.minimum(t, _JW - 1))),
                pl.BlockSpec((1, D, _BW),
                             lambda t, idx, w: (idx[0, 1], 0,
                                                jnp.minimum(t, _JW - 1))),
                pl.BlockSpec((1, 1, D), lambda t, idx, w: (idx[0, 0], 0, 0)),
                pl.BlockSpec((1, 1, D), lambda t, idx, w: (idx[0, 1], 0, 0)),
            ],
            out_specs=pl.BlockSpec((_BN, D),
                                   lambda t, idx, w: (jnp.maximum(t - _JW, 0),
                                                      0)),
            scratch_shapes=[pltpu.VMEM((D, D), jnp.bfloat16),
                            pltpu.VMEM((1, D), jnp.float32)],
        ),
        out_shape=jax.ShapeDtypeStruct((N, D), jnp.float32),
        compiler_params=pltpu.CompilerParams(
            dimension_semantics=("arbitrary",),
        ),
    )(idx8, w8, x, experts_W, experts_W,
      experts_b.reshape(E, 1, D), experts_b.reshape(E, 1, D))
    return out
